# Initial kernel scaffold; baseline (speedup 1.0000x reference)
import jax
import jax.numpy as jnp
from jax.experimental import pallas as pl


def kernel(indices, table):
    raise NotImplementedError("write your pallas kernel here")


# speedup vs baseline: 1.4638x; 1.4638x over previous
import functools

import jax
import jax.numpy as jnp
from jax import lax
from jax.experimental import pallas as pl
from jax.experimental.pallas import tpu as pltpu
from jax.experimental.pallas import tpu_sc as plsc

VOCAB = 1000000
EMBED = 32
BATCH = 4096
SEQ = 200

NC = 2
NS = 16
NW = NC * NS
B_TOTAL = BATCH * SEQ
PER_W = B_TOTAL // NW
CHUNK = 1024
N_CHUNKS = PER_W // CHUNK


@functools.partial(
    pl.kernel,
    out_type=jax.ShapeDtypeStruct((B_TOTAL, EMBED), jnp.float32),
    mesh=plsc.VectorSubcoreMesh(
        core_axis_name="c", subcore_axis_name="s",
        num_cores=NC, num_subcores=NS),
    scratch_types=[
        pltpu.VMEM((CHUNK,), jnp.int32),
        pltpu.VMEM((CHUNK, EMBED), jnp.float32),
        pltpu.SemaphoreType.DMA,
    ],
    compiler_params=pltpu.CompilerParams(use_tc_tiling_on_sc=False),
)
def _gather_kernel(table_hbm, idx_hbm, out_hbm, idx_v, rows_v, sem):
    wid = lax.axis_index("s") * NC + lax.axis_index("c")
    base = wid * PER_W

    @pl.loop(0, N_CHUNKS)
    def _chunk(j):
        off = base + j * CHUNK
        pltpu.sync_copy(idx_hbm.at[pl.ds(off, CHUNK)], idx_v)
        pltpu.async_copy(table_hbm.at[idx_v], rows_v, sem).wait()
        pltpu.sync_copy(rows_v, out_hbm.at[pl.ds(off, CHUNK)])


def kernel(indices, table):
    flat = indices.reshape(B_TOTAL)
    out = _gather_kernel(table, flat)
    return out.reshape(BATCH, SEQ, EMBED)


# speedup vs baseline: 1.4927x; 1.0198x over previous
import functools

import jax
import jax.numpy as jnp
from jax import lax
from jax.experimental import pallas as pl
from jax.experimental.pallas import tpu as pltpu
from jax.experimental.pallas import tpu_sc as plsc

VOCAB = 1000000
EMBED = 32
BATCH = 4096
SEQ = 200

NC = 2
NS = 16
NW = NC * NS
B_TOTAL = BATCH * SEQ
PER_W = B_TOTAL // NW
CHUNK = 1600
N_CHUNKS = PER_W // CHUNK
N_PAIRS = N_CHUNKS // 2


@functools.partial(
    pl.kernel,
    out_type=jax.ShapeDtypeStruct((B_TOTAL, EMBED), jnp.float32),
    mesh=plsc.VectorSubcoreMesh(
        core_axis_name="c", subcore_axis_name="s",
        num_cores=NC, num_subcores=NS),
    scratch_types=[
        pltpu.VMEM((CHUNK,), jnp.int32),
        pltpu.VMEM((CHUNK,), jnp.int32),
        pltpu.VMEM((CHUNK, EMBED), jnp.float32),
        pltpu.VMEM((CHUNK, EMBED), jnp.float32),
        pltpu.SemaphoreType.DMA,
        pltpu.SemaphoreType.DMA,
        pltpu.SemaphoreType.DMA,
        pltpu.SemaphoreType.DMA,
        pltpu.SemaphoreType.DMA,
        pltpu.SemaphoreType.DMA,
    ],
    compiler_params=pltpu.CompilerParams(use_tc_tiling_on_sc=False),
)
def _gather_kernel(table_hbm, idx_hbm, out_hbm,
                   i0, i1, r0, r1, si0, si1, sg0, sg1, so0, so1):
    idx_v = [i0, i1]
    rows_v = [r0, r1]
    isem = [si0, si1]
    gsem = [sg0, sg1]
    osem = [so0, so1]

    wid = lax.axis_index("s") * NC + lax.axis_index("c")
    base = wid * PER_W

    def idx_start(c, p):
        pltpu.async_copy(idx_hbm.at[pl.ds(base + c * CHUNK, CHUNK)],
                         idx_v[p], isem[p])

    idx_start(0, 0)

    @pl.loop(0, N_PAIRS)
    def _pair(jj):
        for p in range(2):
            c = jj * 2 + p
            off = base + c * CHUNK
            pltpu.make_async_copy(
                idx_hbm.at[pl.ds(off, CHUNK)], idx_v[p], isem[p]).wait()

            @pl.when(jj > 0)
            def _():
                pltpu.make_async_copy(
                    rows_v[p],
                    out_hbm.at[pl.ds(off - 2 * CHUNK, CHUNK)],
                    osem[p]).wait()

            gather = pltpu.async_copy(table_hbm.at[idx_v[p]], rows_v[p],
                                      gsem[p])

            if p == 0:
                idx_start(c + 1, 1)
            else:
                @pl.when(jj < N_PAIRS - 1)
                def _():
                    idx_start(c + 1, 0)

            gather.wait()
            pltpu.async_copy(rows_v[p], out_hbm.at[pl.ds(off, CHUNK)],
                             osem[p])

    for p in range(2):
        c = N_CHUNKS - 2 + p
        pltpu.make_async_copy(
            rows_v[p], out_hbm.at[pl.ds(base + c * CHUNK, CHUNK)],
            osem[p]).wait()


def kernel(indices, table):
    flat = indices.reshape(B_TOTAL)
    out = _gather_kernel(table, flat)
    return out.reshape(BATCH, SEQ, EMBED)


# speedup vs baseline: 1.4950x; 1.0016x over previous
import functools

import jax
import jax.numpy as jnp
from jax import lax
from jax.experimental import pallas as pl
from jax.experimental.pallas import tpu as pltpu
from jax.experimental.pallas import tpu_sc as plsc

VOCAB = 1000000
EMBED = 32
BATCH = 4096
SEQ = 200

NC = 2
NS = 16
NW = NC * NS
B_TOTAL = BATCH * SEQ
PER_W = B_TOTAL // NW
CHUNK = 1600
N_CHUNKS = PER_W // CHUNK
N_PAIRS = N_CHUNKS // 2
N_SUB = 4


@functools.partial(
    pl.kernel,
    out_type=jax.ShapeDtypeStruct((B_TOTAL, EMBED), jnp.float32),
    mesh=plsc.VectorSubcoreMesh(
        core_axis_name="c", subcore_axis_name="s",
        num_cores=NC, num_subcores=NS),
    scratch_types=[
        pltpu.VMEM((CHUNK,), jnp.int32),
        pltpu.VMEM((CHUNK,), jnp.int32),
        pltpu.VMEM((CHUNK, EMBED), jnp.float32),
        pltpu.VMEM((CHUNK, EMBED), jnp.float32),
        pltpu.SemaphoreType.DMA,
        pltpu.SemaphoreType.DMA,
        pltpu.SemaphoreType.DMA,
        pltpu.SemaphoreType.DMA,
        pltpu.SemaphoreType.DMA,
        pltpu.SemaphoreType.DMA,
    ],
    compiler_params=pltpu.CompilerParams(use_tc_tiling_on_sc=False),
)
def _gather_kernel(table_hbm, idx_hbm, out_hbm,
                   i0, i1, r0, r1, si0, si1, sg0, sg1, so0, so1):
    idx_v = [i0, i1]
    rows_v = [r0, r1]
    isem = [si0, si1]
    gsem = [sg0, sg1]
    osem = [so0, so1]

    wid = lax.axis_index("s") * NC + lax.axis_index("c")
    base = wid * PER_W

    def idx_start(c, p):
        pltpu.async_copy(idx_hbm.at[pl.ds(base + c * CHUNK, CHUNK)],
                         idx_v[p], isem[p])

    idx_start(0, 0)

    @pl.loop(0, N_PAIRS)
    def _pair(jj):
        for p in range(2):
            c = jj * 2 + p
            off = base + c * CHUNK
            pltpu.make_async_copy(
                idx_hbm.at[pl.ds(off, CHUNK)], idx_v[p], isem[p]).wait()

            @pl.when(jj > 0)
            def _():
                pltpu.make_async_copy(
                    rows_v[p],
                    out_hbm.at[pl.ds(off - 2 * CHUNK, CHUNK)],
                    osem[p]).wait()

            sub = CHUNK // N_SUB
            for q in range(N_SUB):
                pltpu.async_copy(
                    table_hbm.at[idx_v[p].at[pl.ds(q * sub, sub)]],
                    rows_v[p].at[pl.ds(q * sub, sub)],
                    gsem[p])

            if p == 0:
                idx_start(c + 1, 1)
            else:
                @pl.when(jj < N_PAIRS - 1)
                def _():
                    idx_start(c + 1, 0)

            for q in range(N_SUB):
                pltpu.make_async_copy(
                    table_hbm.at[idx_v[p].at[pl.ds(0, CHUNK // N_SUB)]],
                    rows_v[p].at[pl.ds(0, CHUNK // N_SUB)],
                    gsem[p]).wait()
            pltpu.async_copy(rows_v[p], out_hbm.at[pl.ds(off, CHUNK)],
                             osem[p])

    for p in range(2):
        c = N_CHUNKS - 2 + p
        pltpu.make_async_copy(
            rows_v[p], out_hbm.at[pl.ds(base + c * CHUNK, CHUNK)],
            osem[p]).wait()


def kernel(indices, table):
    flat = indices.reshape(B_TOTAL)
    out = _gather_kernel(table, flat)
    return out.reshape(BATCH, SEQ, EMBED)
